# baseline (speedup 1.0000x reference)
import functools

import jax
import jax.numpy as jnp
from jax import lax
from jax.experimental import pallas as pl
from jax.experimental.pallas import tpu as pltpu
from jax.experimental.pallas import tpu_sc as plsc


def _table_body(code_ref, gsum_ref, cb_ref, out_ref):
    x = code_ref[...]
    f, vb = x.shape
    nb = gsum_ref.shape[0]
    nk = f // nb
    e = jnp.exp(x)
    s = jnp.sum(e.reshape(nb, nk, vb), axis=1)
    r = 1.0 / s
    rb = jnp.broadcast_to(r[:, None, :], (nb, nk, vb)).reshape(f, vb)
    p = e * rb
    out_ref[...] = jnp.dot(cb_ref[...], p,
                           preferred_element_type=jnp.float32)


def _build_table(code_t, cbt, num_codebook, num_codeword, block_lanes):
    f, v = code_t.shape
    d = cbt.shape[0]
    grid = (pl.cdiv(v, block_lanes),)
    group = jnp.arange(f, dtype=jnp.int32) // num_codeword
    gsum = (jnp.arange(num_codebook)[:, None] == group[None, :]
            ).astype(jnp.float32)
    return pl.pallas_call(
        _table_body,
        grid=grid,
        in_specs=[
            pl.BlockSpec((f, block_lanes), lambda i: (0, i)),
            pl.BlockSpec((num_codebook, f), lambda i: (0, 0)),
            pl.BlockSpec((d, f), lambda i: (0, 0)),
        ],
        out_specs=pl.BlockSpec((d, block_lanes), lambda i: (0, i)),
        out_shape=jax.ShapeDtypeStruct((d, v), jnp.float32),
    )(code_t, gsum, cbt)


def _make_sc_gather_t(v, d, w, n):
    info = plsc.get_sparse_core_info()
    nc, ns = info.num_cores, info.num_subcores
    nw = nc * ns
    assert n % nw == 0
    nt = n // nw
    assert nt % 16 == 0
    mesh = plsc.VectorSubcoreMesh(core_axis_name="c", subcore_axis_name="s")

    @functools.partial(
        pl.kernel,
        mesh=mesh,
        out_type=jax.ShapeDtypeStruct((w, d, n), jnp.float32),
        scratch_types=[
            pltpu.VMEM((w, nt), jnp.int32),
            pltpu.VMEM((10, nt, d), jnp.float32),
            pltpu.VMEM((5, d, nt), jnp.float32),
            pltpu.SemaphoreType.DMA,
        ],
        compiler_params=pltpu.CompilerParams(use_tc_tiling_on_sc=False,
                                             needs_layout_passes=False),
    )
    def gather(table_hbm, idx_hbm, out_hbm, idx_v, rows_v, col_v, sem):
        wid = lax.axis_index("s") * nc + lax.axis_index("c")
        base = wid * nt
        pltpu.sync_copy(idx_hbm.at[:, pl.ds(base, nt)], idx_v)
        lane = lax.iota(jnp.int32, 16)
        cw = 5
        nch = w // cw

        def fire(c):
            par = jnp.bitwise_and(c, 1)
            for j in range(cw):
                pltpu.async_copy(table_hbm.at[idx_v.at[c * cw + j]],
                                 rows_v.at[par * cw + j], sem)

        def body(c, carry):
            @pl.when(c + 1 < nch)
            def _():
                fire(c + 1)
            for j in range(cw):
                pltpu.make_async_copy(table_hbm.at[idx_v.at[0]],
                                      rows_v.at[0], sem).wait()
            par = jnp.bitwise_and(c, 1)
            parv = jnp.full((16,), par * cw, jnp.int32)
            for j in range(cw):
                slotv = parv + j
                for di in range(d):
                    dvec = jnp.full((16,), di, jnp.int32)
                    for li in range(nt // 16):
                        vals = plsc.load_gather(
                            rows_v, [slotv, li * 16 + lane, dvec])
                        col_v[j, di, pl.ds(li * 16, 16)] = vals
            pltpu.sync_copy(col_v,
                            out_hbm.at[pl.ds(c * cw, cw), :, pl.ds(base, nt)])
            return carry

        fire(0)
        lax.fori_loop(0, nch, body, 0)

    return gather


def kernel(input, code, codebook):
    batch, w = input.shape
    v, num_codebook, num_codeword = code.shape
    d = codebook.shape[-1]
    f = num_codebook * num_codeword

    code_t = code.transpose(1, 2, 0).reshape(f, v)
    cbt = codebook.transpose(2, 0, 1).reshape(d, f)
    table = _build_table(code_t, cbt, num_codebook, num_codeword,
                         block_lanes=2048).T

    idx_t = input.T.astype(jnp.int32)
    out_t = _make_sc_gather_t(v, d, w, batch)(table, idx_t)
    return out_t.transpose(2, 0, 1)


# speedup vs baseline: 1.0502x; 1.0502x over previous
import functools

import jax
import jax.numpy as jnp
from jax import lax
from jax.experimental import pallas as pl
from jax.experimental.pallas import tpu as pltpu
from jax.experimental.pallas import tpu_sc as plsc


def _table_body(code_ref, gsum_ref, cb_ref, out_ref):
    x = code_ref[...]
    f, vb = x.shape
    nb = gsum_ref.shape[0]
    nk = f // nb
    e = jnp.exp(x)
    s = jnp.sum(e.reshape(nb, nk, vb), axis=1)
    r = 1.0 / s
    rb = jnp.broadcast_to(r[:, None, :], (nb, nk, vb)).reshape(f, vb)
    p = e * rb
    out_ref[...] = jax.lax.dot_general(
        p, cb_ref[...], (((0,), (0,)), ((), ())),
        preferred_element_type=jnp.float32)


def _build_table(code_t, cb2d, num_codebook, num_codeword, block_lanes):
    f, v = code_t.shape
    d = cb2d.shape[1]
    grid = (pl.cdiv(v, block_lanes),)
    group = jnp.arange(f, dtype=jnp.int32) // num_codeword
    gsum = (jnp.arange(num_codebook)[:, None] == group[None, :]
            ).astype(jnp.float32)
    return pl.pallas_call(
        _table_body,
        grid=grid,
        in_specs=[
            pl.BlockSpec((f, block_lanes), lambda i: (0, i)),
            pl.BlockSpec((num_codebook, f), lambda i: (0, 0)),
            pl.BlockSpec((f, d), lambda i: (0, 0)),
        ],
        out_specs=pl.BlockSpec((block_lanes, d), lambda i: (i, 0)),
        out_shape=jax.ShapeDtypeStruct((v, d), jnp.float32),
    )(code_t, gsum, cb2d)


def _make_sc_gather_t(v, d, w, n):
    info = plsc.get_sparse_core_info()
    nc, ns = info.num_cores, info.num_subcores
    nw = nc * ns
    assert n % nw == 0
    nt = n // nw
    assert nt % 16 == 0
    mesh = plsc.VectorSubcoreMesh(core_axis_name="c", subcore_axis_name="s")

    @functools.partial(
        pl.kernel,
        mesh=mesh,
        out_type=jax.ShapeDtypeStruct((w, d, n), jnp.float32),
        scratch_types=[
            pltpu.VMEM((w, nt), jnp.int32),
            pltpu.VMEM((10, nt, d), jnp.float32),
            pltpu.VMEM((10, d, nt), jnp.float32),
            pltpu.SemaphoreType.DMA,
            pltpu.SemaphoreType.DMA,
        ],
        compiler_params=pltpu.CompilerParams(use_tc_tiling_on_sc=False,
                                             needs_layout_passes=False),
    )
    def gather(table_hbm, idx_hbm, out_hbm, idx_v, rows_v, col_v, sem, sem_o):
        wid = lax.axis_index("s") * nc + lax.axis_index("c")
        base = wid * nt
        pltpu.sync_copy(idx_hbm.at[:, pl.ds(base, nt)], idx_v)
        lane = lax.iota(jnp.int32, 16)
        cw = 5
        nch = w // cw

        def fire(c):
            par = jnp.bitwise_and(c, 1)
            for j in range(cw):
                pltpu.async_copy(table_hbm.at[idx_v.at[c * cw + j]],
                                 rows_v.at[par * cw + j], sem)

        def drain_out():
            pltpu.make_async_copy(
                out_hbm.at[pl.ds(0, cw), :, pl.ds(base, nt)],
                col_v.at[pl.ds(0, cw)], sem_o).wait()

        def body(c, carry):
            @pl.when(c + 1 < nch)
            def _():
                fire(c + 1)
            for j in range(cw):
                pltpu.make_async_copy(table_hbm.at[idx_v.at[0]],
                                      rows_v.at[0], sem).wait()
            @pl.when(c >= 2)
            def _():
                drain_out()
            par = jnp.bitwise_and(c, 1)
            parv = jnp.full((16,), par * cw, jnp.int32)
            for j in range(cw):
                slotv = parv + j
                for di in range(d):
                    dvec = jnp.full((16,), di, jnp.int32)
                    for li in range(nt // 16):
                        vals = plsc.load_gather(
                            rows_v, [slotv, li * 16 + lane, dvec])
                        col_v[par * cw + j, di, pl.ds(li * 16, 16)] = vals
            pltpu.async_copy(col_v.at[pl.ds(par * cw, cw)],
                             out_hbm.at[pl.ds(c * cw, cw), :, pl.ds(base, nt)],
                             sem_o)
            return carry

        fire(0)
        lax.fori_loop(0, nch, body, 0)
        drain_out()
        drain_out()

    return gather


def kernel(input, code, codebook):
    batch, w = input.shape
    v, num_codebook, num_codeword = code.shape
    d = codebook.shape[-1]
    f = num_codebook * num_codeword

    code_t = code.transpose(1, 2, 0).reshape(f, v)
    cb2d = codebook.reshape(f, d)
    table = _build_table(code_t, cb2d, num_codebook, num_codeword,
                         block_lanes=2048)

    idx_t = input.T.astype(jnp.int32)
    out_t = _make_sc_gather_t(v, d, w, batch)(table, idx_t)
    return out_t.transpose(2, 0, 1)
